# baseline (speedup 1.0000x reference)
import functools

import jax
import jax.numpy as jnp
from jax import lax
from jax.experimental import pallas as pl
from jax.experimental.pallas import tpu as pltpu
from jax.experimental.pallas import tpu_sc as plsc

_T = 4096
_D = 1024
_NC = 2
_NS = 16
_NW = _NC * _NS
_SPLIT = _T // 2
_RPW = (_T - _SPLIT) // _NW
_CH = 32
_NBUF = 2


def _make_sc_copy():
    mesh = plsc.VectorSubcoreMesh(core_axis_name="c", subcore_axis_name="s")
    n = _RPW // _CH

    @functools.partial(
        pl.kernel,
        mesh=mesh,
        out_type=jax.ShapeDtypeStruct((_T, _D), jnp.float32),
        scratch_types=[
            pltpu.VMEM_SHARED((_NS, _NBUF, _CH, _D), jnp.float32),
            *([pltpu.SemaphoreType.DMA] * (2 * _NBUF)),
        ],
    )
    def sc_copy(table_hbm, out_hbm, stage, *sems):
        in_sems = sems[:_NBUF]
        out_sems = sems[_NBUF:]
        wid = lax.axis_index("s") * _NC + lax.axis_index("c")
        sid = lax.axis_index("s")
        base = _SPLIT + wid * _RPW

        def fire_in(k):
            b = k % _NBUF
            return pltpu.async_copy(
                table_hbm.at[pl.ds(base + k * _CH, _CH)], stage.at[sid, b], in_sems[b]
            )

        def fire_out(k):
            b = k % _NBUF
            return pltpu.async_copy(
                stage.at[sid, b], out_hbm.at[pl.ds(base + k * _CH, _CH)], out_sems[b]
            )

        in_cp = [None] * n
        out_cp = [None] * n
        for j in range(min(_NBUF, n)):
            in_cp[j] = fire_in(j)
        for k in range(n):
            if k >= _NBUF:
                out_cp[k - _NBUF].wait()
                in_cp[k] = fire_in(k)
            in_cp[k].wait()
            out_cp[k] = fire_out(k)
        for k in range(max(0, n - _NBUF), n):
            out_cp[k].wait()

    return sc_copy


_sc_copy = _make_sc_copy()

_TCB = 256


def _tc_fill(pe_table, sc_out):
    def body(table_ref, alias_ref, out_ref):
        del alias_ref
        out_ref[...] = table_ref[...]

    return pl.pallas_call(
        body,
        grid=(_SPLIT // _TCB,),
        in_specs=[
            pl.BlockSpec((_TCB, _D), lambda i: (i, 0)),
            pl.BlockSpec(memory_space=pl.ANY),
        ],
        out_specs=pl.BlockSpec((_TCB, _D), lambda i: (i, 0)),
        out_shape=jax.ShapeDtypeStruct((_T, _D), jnp.float32),
        input_output_aliases={1: 0},
    )(pe_table, sc_out)


def kernel(x, pe_table):
    del x
    sc_out = _sc_copy(pe_table)
    out = _tc_fill(pe_table, sc_out)
    return out[None]


# speedup vs baseline: 1.0157x; 1.0157x over previous
import functools

import jax
import jax.numpy as jnp
from jax import lax
from jax.experimental import pallas as pl
from jax.experimental.pallas import tpu as pltpu
from jax.experimental.pallas import tpu_sc as plsc

_T = 4096
_D = 1024
_NC = 2
_NS = 16
_NW = _NC * _NS
_SC_ROWS = 1024
_SPLIT = _T - _SC_ROWS
_RPW = _SC_ROWS // _NW
_CH = 16
_NBUF = 2


def _make_sc_copy():
    mesh = plsc.VectorSubcoreMesh(core_axis_name="c", subcore_axis_name="s")
    n = _RPW // _CH

    @functools.partial(
        pl.kernel,
        mesh=mesh,
        out_type=jax.ShapeDtypeStruct((_SC_ROWS, _D), jnp.float32),
        scratch_types=[
            pltpu.VMEM_SHARED((_NS, _NBUF, _CH, _D), jnp.float32),
            *([pltpu.SemaphoreType.DMA] * (2 * _NBUF)),
        ],
    )
    def sc_copy(table_hbm, out_hbm, stage, *sems):
        in_sems = sems[:_NBUF]
        out_sems = sems[_NBUF:]
        wid = lax.axis_index("s") * _NC + lax.axis_index("c")
        sid = lax.axis_index("s")
        src = _SPLIT + wid * _RPW
        dst = wid * _RPW

        def fire_in(k):
            b = k % _NBUF
            return pltpu.async_copy(
                table_hbm.at[pl.ds(src + k * _CH, _CH)], stage.at[sid, b], in_sems[b]
            )

        def fire_out(k):
            b = k % _NBUF
            return pltpu.async_copy(
                stage.at[sid, b], out_hbm.at[pl.ds(dst + k * _CH, _CH)], out_sems[b]
            )

        in_cp = [None] * n
        out_cp = [None] * n
        for j in range(min(_NBUF, n)):
            in_cp[j] = fire_in(j)
        for k in range(n):
            if k >= _NBUF:
                out_cp[k - _NBUF].wait()
                in_cp[k] = fire_in(k)
            in_cp[k].wait()
            out_cp[k] = fire_out(k)
        for k in range(max(0, n - _NBUF), n):
            out_cp[k].wait()

    return sc_copy


_sc_copy = _make_sc_copy()

_TCB = 256


def _tc_head(pe_table):
    def body(table_ref, out_ref):
        out_ref[...] = table_ref[...]

    return pl.pallas_call(
        body,
        grid=(_SPLIT // _TCB,),
        in_specs=[pl.BlockSpec((_TCB, _D), lambda i: (i, 0))],
        out_specs=pl.BlockSpec((_TCB, _D), lambda i: (i, 0)),
        out_shape=jax.ShapeDtypeStruct((_T, _D), jnp.float32),
    )(pe_table)


def _tc_stitch(sc_buf, head_out):
    def body(sc_ref, alias_ref, out_ref):
        del alias_ref
        out_ref[...] = sc_ref[...]

    off = _SPLIT // _TCB
    return pl.pallas_call(
        body,
        grid=(_SC_ROWS // _TCB,),
        in_specs=[
            pl.BlockSpec((_TCB, _D), lambda i: (i, 0)),
            pl.BlockSpec(memory_space=pl.ANY),
        ],
        out_specs=pl.BlockSpec((_TCB, _D), lambda i: (i + off, 0)),
        out_shape=jax.ShapeDtypeStruct((_T, _D), jnp.float32),
        input_output_aliases={1: 0},
    )(sc_buf, head_out)


def kernel(x, pe_table):
    del x
    sc_buf = _sc_copy(pe_table)
    head = _tc_head(pe_table)
    out = _tc_stitch(sc_buf, head)
    return out[None]


# speedup vs baseline: 1.1733x; 1.1552x over previous
import functools

import jax
import jax.numpy as jnp
from jax import lax
from jax.experimental import pallas as pl
from jax.experimental.pallas import tpu as pltpu
from jax.experimental.pallas import tpu_sc as plsc

_T = 4096
_D = 1024
_NC = 2
_NS = 16
_NW = _NC * _NS
_RPW = _T // _NW
_HALF = _RPW // 2
_CH = 32
_NBUF = 2
_NCH = _HALF // _CH


def _make_sc_copy():
    mesh = plsc.VectorSubcoreMesh(core_axis_name="c", subcore_axis_name="s")

    @functools.partial(
        pl.kernel,
        mesh=mesh,
        out_type=jax.ShapeDtypeStruct((_T, _D), jnp.float32),
        scratch_types=[
            pltpu.VMEM((_NBUF, _CH, _D), jnp.float32),
            pltpu.VMEM_SHARED((_NS, _NBUF, _CH, _D), jnp.float32),
            *([pltpu.SemaphoreType.DMA] * (4 * _NBUF)),
        ],
    )
    def sc_copy(table_hbm, out_hbm, tbuf, sbuf, *sems):
        t_in = sems[0:_NBUF]
        t_out = sems[_NBUF : 2 * _NBUF]
        s_in = sems[2 * _NBUF : 3 * _NBUF]
        s_out = sems[3 * _NBUF : 4 * _NBUF]
        wid = lax.axis_index("s") * _NC + lax.axis_index("c")
        sid = lax.axis_index("s")
        base_t = wid * _RPW
        base_s = base_t + _HALF

        def t_fire_in(k):
            b = k % _NBUF
            return pltpu.async_copy(
                table_hbm.at[pl.ds(base_t + k * _CH, _CH)], tbuf.at[b], t_in[b]
            )

        def t_fire_out(k):
            b = k % _NBUF
            return pltpu.async_copy(
                tbuf.at[b], out_hbm.at[pl.ds(base_t + k * _CH, _CH)], t_out[b]
            )

        def s_fire_in(k):
            b = k % _NBUF
            return pltpu.async_copy(
                table_hbm.at[pl.ds(base_s + k * _CH, _CH)], sbuf.at[sid, b], s_in[b]
            )

        def s_fire_out(k):
            b = k % _NBUF
            return pltpu.async_copy(
                sbuf.at[sid, b], out_hbm.at[pl.ds(base_s + k * _CH, _CH)], s_out[b]
            )

        t_icp = [None] * _NCH
        t_ocp = [None] * _NCH
        s_icp = [None] * _NCH
        s_ocp = [None] * _NCH
        for j in range(min(_NBUF, _NCH)):
            t_icp[j] = t_fire_in(j)
            s_icp[j] = s_fire_in(j)
        for k in range(_NCH):
            if k >= _NBUF:
                t_ocp[k - _NBUF].wait()
                t_icp[k] = t_fire_in(k)
                s_ocp[k - _NBUF].wait()
                s_icp[k] = s_fire_in(k)
            t_icp[k].wait()
            t_ocp[k] = t_fire_out(k)
            s_icp[k].wait()
            s_ocp[k] = s_fire_out(k)
        for k in range(max(0, _NCH - _NBUF), _NCH):
            t_ocp[k].wait()
            s_ocp[k].wait()

    return sc_copy


_sc_copy = _make_sc_copy()


def kernel(x, pe_table):
    del x
    out = _sc_copy(pe_table)
    return out[None]
